# Initial kernel scaffold; baseline (speedup 1.0000x reference)
import jax
import jax.numpy as jnp
from jax.experimental import pallas as pl


def kernel(x, edge_index, batch, W1a, b1a, g1, be1, W1b, b1b, W2a, b2a, g2, be2, W2b, b2b, Wl, bl):
    raise NotImplementedError("write your pallas kernel here")


# speedup vs baseline: 4.4465x; 4.4465x over previous
import functools

import jax
import jax.numpy as jnp
from jax import lax
from jax.experimental import pallas as pl
from jax.experimental.pallas import tpu as pltpu
from jax.experimental.pallas import tpu_sc as plsc

N = 10000
E = 160000
D_IN = 768
HID = 256
G = 64
BN_EPS = 1e-5

NC = 2
NS = 16
CK = 128
CHUNKS = 80
E_PAD = NC * NS * CHUNKS * CK // 2
EPT = CHUNKS * CK
N_PAD = 10112
ZR = N_PAD // NS

BNODE = 400
NB = N // BNODE

HALF = HID // 2


def _sc_segment_sum(y2, src_w, dst_r):
    y_flat = y2.reshape(NC * N, HALF)

    mesh = plsc.VectorSubcoreMesh(
        core_axis_name="c", subcore_axis_name="s", num_cores=NC, num_subcores=NS
    )

    @functools.partial(
        pl.kernel,
        out_type=jax.ShapeDtypeStruct((NC * N_PAD, HALF), jnp.float32),
        mesh=mesh,
        scratch_types=[
            pltpu.VMEM((CHUNKS, CK), jnp.int32),
            pltpu.VMEM((CHUNKS, CK), jnp.int32),
            pltpu.VMEM((CK, HALF), jnp.float32),
            pltpu.VMEM_SHARED((N_PAD, HALF), jnp.float32),
            pltpu.SemaphoreType.DMA,
        ],
    )
    def k(y_hbm, src_hbm, dst_hbm, out_hbm, src_v, dst_v, rows_a,
          acc, sem_a):
        c = lax.axis_index("c")
        s = lax.axis_index("s")
        w = c * NS + s

        pltpu.sync_copy(src_hbm.at[w], src_v)
        pltpu.sync_copy(dst_hbm.at[s], dst_v)

        def zero_row(i, _):
            for j in range(HALF // 16):
                rows_a[i, pl.ds(j * 16, 16)] = jnp.zeros((16,), jnp.float32)
            return 0
        lax.fori_loop(0, CK, zero_row, 0)
        base = s * ZR
        nfull = ZR // CK
        for k4 in range(nfull):
            pltpu.sync_copy(rows_a, acc.at[pl.ds(base + k4 * CK, CK)])
        rem = ZR - nfull * CK
        if rem:
            pltpu.sync_copy(rows_a.at[pl.ds(0, rem)],
                            acc.at[pl.ds(base + nfull * CK, rem)])
        plsc.subcore_barrier()

        def body(j, _):
            pltpu.async_copy(y_hbm.at[src_v.at[j]], rows_a, sem_a).wait()
            pltpu.sync_copy(rows_a, acc.at[dst_v.at[j]], add=True)
            return 0

        lax.fori_loop(0, CHUNKS, body, 0)
        plsc.subcore_barrier()

        pltpu.sync_copy(acc.at[pl.ds(base, ZR)],
                        out_hbm.at[pl.ds(c * N_PAD + base, ZR)])

    out = k(y_flat, src_w, dst_r)
    return out.reshape(NC, N_PAD, HALF)


def _k1_body(x_ref, w_ref, y_ref):
    r = jnp.dot(x_ref[...], w_ref[...], preferred_element_type=jnp.float32)
    y_ref[0] = r[:, :HALF]
    y_ref[1] = r[:, HALF:]


def _k1(x, W1a):
    return pl.pallas_call(
        _k1_body,
        grid=(NB,),
        in_specs=[
            pl.BlockSpec((BNODE, D_IN), lambda i: (i, 0)),
            pl.BlockSpec((D_IN, HID), lambda i: (0, 0)),
        ],
        out_specs=pl.BlockSpec((NC, BNODE, HALF), lambda i: (0, i, 0)),
        out_shape=jax.ShapeDtypeStruct((NC, N, HALF), jnp.float32),
    )(x, W1a)


def _k2_body(y_ref, a_ref, s1_ref, c1_ref, w1b_ref, b1b_ref, w2a_ref,
             x1_ref, y2_ref):
    h = jnp.concatenate([y_ref[0] + a_ref[0], y_ref[1] + a_ref[1]], axis=1)
    h = jnp.maximum(h * s1_ref[...] + c1_ref[...], 0.0)
    x1 = jnp.dot(h, w1b_ref[...], preferred_element_type=jnp.float32)
    x1 = jnp.maximum(x1 + b1b_ref[...], 0.0)
    x1_ref[...] = x1
    r2 = jnp.dot(x1, w2a_ref[...], preferred_element_type=jnp.float32)
    y2_ref[0] = r2[:, :HALF]
    y2_ref[1] = r2[:, HALF:]


def _k2(y, agg, s1, c1, W1b, b1b, W2a):
    return pl.pallas_call(
        _k2_body,
        grid=(NB,),
        in_specs=[
            pl.BlockSpec((NC, BNODE, HALF), lambda i: (0, i, 0)),
            pl.BlockSpec((NC, BNODE, HALF), lambda i: (0, i, 0)),
            pl.BlockSpec((1, HID), lambda i: (0, 0)),
            pl.BlockSpec((1, HID), lambda i: (0, 0)),
            pl.BlockSpec((HID, HID), lambda i: (0, 0)),
            pl.BlockSpec((1, HID), lambda i: (0, 0)),
            pl.BlockSpec((HID, HID), lambda i: (0, 0)),
        ],
        out_specs=[
            pl.BlockSpec((BNODE, HID), lambda i: (i, 0)),
            pl.BlockSpec((NC, BNODE, HALF), lambda i: (0, i, 0)),
        ],
        out_shape=[
            jax.ShapeDtypeStruct((N, HID), jnp.float32),
            jax.ShapeDtypeStruct((NC, N, HALF), jnp.float32),
        ],
    )(y, agg, s1, c1, W1b, b1b, W2a)


def _k3_body(y_ref, a_ref, s2_ref, c2_ref, w2b_ref, b2b_ref, x1_ref,
             wl_ref, bl_ref, batch_ref, out_ref):
    i = pl.program_id(0)
    h = jnp.concatenate([y_ref[0] + a_ref[0], y_ref[1] + a_ref[1]], axis=1)
    h = jnp.maximum(h * s2_ref[...] + c2_ref[...], 0.0)
    x2 = jnp.dot(h, w2b_ref[...], preferred_element_type=jnp.float32)
    x2 = jnp.maximum(x2 + b2b_ref[...], 0.0)
    p = (jnp.dot(x1_ref[...], wl_ref[:HID], preferred_element_type=jnp.float32)
         + jnp.dot(x2, wl_ref[HID:], preferred_element_type=jnp.float32))
    b = batch_ref[0, 0, :]
    onehot = (b[None, :] == lax.broadcasted_iota(jnp.int32, (G, BNODE), 0)
              ).astype(jnp.float32)
    contrib = jnp.dot(onehot, p, preferred_element_type=jnp.float32)

    @pl.when(i == 0)
    def _():
        out_ref[...] = jnp.broadcast_to(bl_ref[...], (G, 2))

    out_ref[...] += contrib


def _k3(y2, agg2, s2, c2, W2b, b2b, x1, Wl, bl, batch3):
    return pl.pallas_call(
        _k3_body,
        grid=(NB,),
        in_specs=[
            pl.BlockSpec((NC, BNODE, HALF), lambda i: (0, i, 0)),
            pl.BlockSpec((NC, BNODE, HALF), lambda i: (0, i, 0)),
            pl.BlockSpec((1, HID), lambda i: (0, 0)),
            pl.BlockSpec((1, HID), lambda i: (0, 0)),
            pl.BlockSpec((HID, HID), lambda i: (0, 0)),
            pl.BlockSpec((1, HID), lambda i: (0, 0)),
            pl.BlockSpec((BNODE, HID), lambda i: (i, 0)),
            pl.BlockSpec((2 * HID, 2), lambda i: (0, 0)),
            pl.BlockSpec((1, 2), lambda i: (0, 0)),
            pl.BlockSpec((1, 1, BNODE), lambda i: (i, 0, 0)),
        ],
        out_specs=pl.BlockSpec((G, 2), lambda i: (0, 0)),
        out_shape=jax.ShapeDtypeStruct((G, 2), jnp.float32),
    )(y2, agg2, s2, c2, W2b, b2b, x1, Wl, bl, batch3)


def kernel(x, edge_index, batch, W1a, b1a, g1, be1, W1b, b1b,
           W2a, b2a, g2, be2, W2b, b2b, Wl, bl):
    src = edge_index[0]
    dst = edge_index[1]

    pad = E_PAD - E
    src_p = jnp.concatenate([src, jnp.zeros((pad,), jnp.int32)])
    dst_p = jnp.concatenate([dst, jnp.full((pad,), N, jnp.int32)])
    src_r = src_p.reshape(NS, CHUNKS, CK)
    src_w = jnp.concatenate([src_r, src_r + N], axis=0).reshape(
        NC * NS, CHUNKS, CK)
    dst_r = dst_p.reshape(NS, CHUNKS, CK)
    batch3 = batch.reshape(NB, 1, BNODE)

    inv = 1.0 / jnp.sqrt(1.0 + BN_EPS)
    s1 = (g1 * inv).reshape(1, HID)
    c1 = (b1a * g1 * inv + be1).reshape(1, HID)
    s2 = (g2 * inv).reshape(1, HID)
    c2 = (b2a * g2 * inv + be2).reshape(1, HID)

    y = _k1(x, W1a)
    agg1 = _sc_segment_sum(y, src_w, dst_r)
    x1, y2 = _k2(y, agg1, s1, c1, W1b, b1b.reshape(1, HID), W2a)
    agg2 = _sc_segment_sum(y2, src_w, dst_r)
    out = _k3(y2, agg2, s2, c2, W2b, b2b.reshape(1, HID), x1,
              Wl, bl.reshape(1, 2), batch3)
    return out


# speedup vs baseline: 4.9344x; 1.1097x over previous
import functools

import jax
import jax.numpy as jnp
from jax import lax
from jax.experimental import pallas as pl
from jax.experimental.pallas import tpu as pltpu
from jax.experimental.pallas import tpu_sc as plsc

N = 10000
E = 160000
D_IN = 768
HID = 256
G = 64
BN_EPS = 1e-5

NC = 2
NS = 16
CK = 128
CHUNKS = 80
PCH = 40
E_PAD = NC * NS * CHUNKS * CK // 2
EPT = CHUNKS * CK
N_PAD = 10112
ZR = N_PAD // NS

BNODE = 400
NB = N // BNODE

HALF = HID // 2


def _sc_segment_sum(y2, src_w, dst_r):
    y_flat = y2.reshape(NC * N, HALF)

    mesh = plsc.VectorSubcoreMesh(
        core_axis_name="c", subcore_axis_name="s", num_cores=NC, num_subcores=NS
    )

    @functools.partial(
        pl.kernel,
        out_type=jax.ShapeDtypeStruct((NC * N_PAD, HALF), jnp.float32),
        mesh=mesh,
        scratch_types=[
            pltpu.VMEM((PCH, CK), jnp.int32),
            pltpu.VMEM((PCH, CK), jnp.int32),
            pltpu.VMEM((CK, HALF), jnp.float32),
            pltpu.VMEM((CK, HALF), jnp.float32),
            pltpu.VMEM_SHARED((N_PAD, HALF), jnp.float32),
            pltpu.SemaphoreType.DMA,
            pltpu.SemaphoreType.DMA,
            pltpu.SemaphoreType.DMA,
            pltpu.SemaphoreType.DMA,
        ],
    )
    def k(y_hbm, src_hbm, dst_hbm, out_hbm, src_v, dst_v, rows_a, rows_b,
          acc, g_a, g_b, s_a, s_b):
        c = lax.axis_index("c")
        s = lax.axis_index("s")
        w = c * NS + s

        def zero_row(i, _):
            for j in range(HALF // 16):
                rows_a[i, pl.ds(j * 16, 16)] = jnp.zeros((16,), jnp.float32)
            return 0
        lax.fori_loop(0, CK, zero_row, 0)
        base = s * ZR
        nfull = ZR // CK
        for k4 in range(nfull):
            pltpu.sync_copy(rows_a, acc.at[pl.ds(base + k4 * CK, CK)])
        rem = ZR - nfull * CK
        if rem:
            pltpu.sync_copy(rows_a.at[pl.ds(0, rem)],
                            acc.at[pl.ds(base + nfull * CK, rem)])
        plsc.subcore_barrier()

        def wait_g(rows, sem):
            pltpu.make_async_copy(y_hbm.at[src_v.at[0]], rows, sem).wait()

        def wait_s(rows, sem):
            pltpu.make_async_copy(rows, acc.at[dst_v.at[0]], sem).wait()

        for h in range(CHUNKS // PCH):
            pltpu.sync_copy(src_hbm.at[w, pl.ds(h * PCH, PCH)], src_v)
            pltpu.sync_copy(dst_hbm.at[s, pl.ds(h * PCH, PCH)], dst_v)
            pltpu.async_copy(y_hbm.at[src_v.at[0]], rows_a, g_a)

            def body(i, _):
                j0 = 2 * i
                j1 = 2 * i + 1
                wait_g(rows_a, g_a)

                @pl.when(i > 0)
                def _():
                    wait_s(rows_b, s_b)

                pltpu.async_copy(y_hbm.at[src_v.at[j1]], rows_b, g_b)
                pltpu.async_copy(rows_a, acc.at[dst_v.at[j0]], s_a, add=True)
                wait_g(rows_b, g_b)

                @pl.when(i < PCH // 2 - 1)
                def _():
                    wait_s(rows_a, s_a)
                    pltpu.async_copy(y_hbm.at[src_v.at[j1 + 1]], rows_a, g_a)

                pltpu.async_copy(rows_b, acc.at[dst_v.at[j1]], s_b, add=True)
                return 0

            lax.fori_loop(0, PCH // 2, body, 0)
            wait_s(rows_a, s_a)
            wait_s(rows_b, s_b)
        plsc.subcore_barrier()

        pltpu.sync_copy(acc.at[pl.ds(base, ZR)],
                        out_hbm.at[pl.ds(c * N_PAD + base, ZR)])

    out = k(y_flat, src_w, dst_r)
    return out.reshape(NC, N_PAD, HALF)


def _k1_body(x_ref, w_ref, y_ref):
    r = jnp.dot(x_ref[...], w_ref[...], preferred_element_type=jnp.float32)
    y_ref[0] = r[:, :HALF]
    y_ref[1] = r[:, HALF:]


def _k1(x, W1a):
    return pl.pallas_call(
        _k1_body,
        grid=(NB,),
        in_specs=[
            pl.BlockSpec((BNODE, D_IN), lambda i: (i, 0)),
            pl.BlockSpec((D_IN, HID), lambda i: (0, 0)),
        ],
        out_specs=pl.BlockSpec((NC, BNODE, HALF), lambda i: (0, i, 0)),
        out_shape=jax.ShapeDtypeStruct((NC, N, HALF), jnp.float32),
    )(x, W1a)


def _k2_body(y_ref, a_ref, s1_ref, c1_ref, w1b_ref, b1b_ref, w2a_ref,
             x1_ref, y2_ref):
    h = jnp.concatenate([y_ref[0] + a_ref[0], y_ref[1] + a_ref[1]], axis=1)
    h = jnp.maximum(h * s1_ref[...] + c1_ref[...], 0.0)
    x1 = jnp.dot(h, w1b_ref[...], preferred_element_type=jnp.float32)
    x1 = jnp.maximum(x1 + b1b_ref[...], 0.0)
    x1_ref[...] = x1
    r2 = jnp.dot(x1, w2a_ref[...], preferred_element_type=jnp.float32)
    y2_ref[0] = r2[:, :HALF]
    y2_ref[1] = r2[:, HALF:]


def _k2(y, agg, s1, c1, W1b, b1b, W2a):
    return pl.pallas_call(
        _k2_body,
        grid=(NB,),
        in_specs=[
            pl.BlockSpec((NC, BNODE, HALF), lambda i: (0, i, 0)),
            pl.BlockSpec((NC, BNODE, HALF), lambda i: (0, i, 0)),
            pl.BlockSpec((1, HID), lambda i: (0, 0)),
            pl.BlockSpec((1, HID), lambda i: (0, 0)),
            pl.BlockSpec((HID, HID), lambda i: (0, 0)),
            pl.BlockSpec((1, HID), lambda i: (0, 0)),
            pl.BlockSpec((HID, HID), lambda i: (0, 0)),
        ],
        out_specs=[
            pl.BlockSpec((BNODE, HID), lambda i: (i, 0)),
            pl.BlockSpec((NC, BNODE, HALF), lambda i: (0, i, 0)),
        ],
        out_shape=[
            jax.ShapeDtypeStruct((N, HID), jnp.float32),
            jax.ShapeDtypeStruct((NC, N, HALF), jnp.float32),
        ],
    )(y, agg, s1, c1, W1b, b1b, W2a)


def _k3_body(y_ref, a_ref, s2_ref, c2_ref, w2b_ref, b2b_ref, x1_ref,
             wl_ref, bl_ref, batch_ref, out_ref):
    i = pl.program_id(0)
    h = jnp.concatenate([y_ref[0] + a_ref[0], y_ref[1] + a_ref[1]], axis=1)
    h = jnp.maximum(h * s2_ref[...] + c2_ref[...], 0.0)
    x2 = jnp.dot(h, w2b_ref[...], preferred_element_type=jnp.float32)
    x2 = jnp.maximum(x2 + b2b_ref[...], 0.0)
    p = (jnp.dot(x1_ref[...], wl_ref[:HID], preferred_element_type=jnp.float32)
         + jnp.dot(x2, wl_ref[HID:], preferred_element_type=jnp.float32))
    b = batch_ref[0, 0, :]
    onehot = (b[None, :] == lax.broadcasted_iota(jnp.int32, (G, BNODE), 0)
              ).astype(jnp.float32)
    contrib = jnp.dot(onehot, p, preferred_element_type=jnp.float32)

    @pl.when(i == 0)
    def _():
        out_ref[...] = jnp.broadcast_to(bl_ref[...], (G, 2))

    out_ref[...] += contrib


def _k3(y2, agg2, s2, c2, W2b, b2b, x1, Wl, bl, batch3):
    return pl.pallas_call(
        _k3_body,
        grid=(NB,),
        in_specs=[
            pl.BlockSpec((NC, BNODE, HALF), lambda i: (0, i, 0)),
            pl.BlockSpec((NC, BNODE, HALF), lambda i: (0, i, 0)),
            pl.BlockSpec((1, HID), lambda i: (0, 0)),
            pl.BlockSpec((1, HID), lambda i: (0, 0)),
            pl.BlockSpec((HID, HID), lambda i: (0, 0)),
            pl.BlockSpec((1, HID), lambda i: (0, 0)),
            pl.BlockSpec((BNODE, HID), lambda i: (i, 0)),
            pl.BlockSpec((2 * HID, 2), lambda i: (0, 0)),
            pl.BlockSpec((1, 2), lambda i: (0, 0)),
            pl.BlockSpec((1, 1, BNODE), lambda i: (i, 0, 0)),
        ],
        out_specs=pl.BlockSpec((G, 2), lambda i: (0, 0)),
        out_shape=jax.ShapeDtypeStruct((G, 2), jnp.float32),
    )(y2, agg2, s2, c2, W2b, b2b, x1, Wl, bl, batch3)


def kernel(x, edge_index, batch, W1a, b1a, g1, be1, W1b, b1b,
           W2a, b2a, g2, be2, W2b, b2b, Wl, bl):
    src = edge_index[0]
    dst = edge_index[1]

    pad = E_PAD - E
    src_p = jnp.concatenate([src, jnp.zeros((pad,), jnp.int32)])
    dst_p = jnp.concatenate([dst, jnp.full((pad,), N, jnp.int32)])
    src_r = src_p.reshape(NS, CHUNKS, CK)
    src_w = jnp.concatenate([src_r, src_r + N], axis=0).reshape(
        NC * NS, CHUNKS, CK)
    dst_r = dst_p.reshape(NS, CHUNKS, CK)
    batch3 = batch.reshape(NB, 1, BNODE)

    inv = 1.0 / jnp.sqrt(1.0 + BN_EPS)
    s1 = (g1 * inv).reshape(1, HID)
    c1 = (b1a * g1 * inv + be1).reshape(1, HID)
    s2 = (g2 * inv).reshape(1, HID)
    c2 = (b2a * g2 * inv + be2).reshape(1, HID)

    y = _k1(x, W1a)
    agg1 = _sc_segment_sum(y, src_w, dst_r)
    x1, y2 = _k2(y, agg1, s1, c1, W1b, b1b.reshape(1, HID), W2a)
    agg2 = _sc_segment_sum(y2, src_w, dst_r)
    out = _k3(y2, agg2, s2, c2, W2b, b2b.reshape(1, HID), x1,
              Wl, bl.reshape(1, 2), batch3)
    return out


# speedup vs baseline: 5.3950x; 1.0933x over previous
import functools

import jax
import jax.numpy as jnp
from jax import lax
from jax.experimental import pallas as pl
from jax.experimental.pallas import tpu as pltpu
from jax.experimental.pallas import tpu_sc as plsc

N = 10000
E = 160000
D_IN = 768
HID = 256
G = 64
BN_EPS = 1e-5

NC = 2
NS = 16
CK = 80
CHUNKS = 128
PCH = 32
NBUF = 4
E_PAD = NC * NS * CHUNKS * CK // 2
EPT = CHUNKS * CK
N_PAD = 10112
ZR = N_PAD // NS

BNODE = 400
NB = N // BNODE

HALF = HID // 2


def _sc_segment_sum(y2, src_w, dst_r):
    y_flat = y2.reshape(NC * N, HALF)

    mesh = plsc.VectorSubcoreMesh(
        core_axis_name="c", subcore_axis_name="s", num_cores=NC, num_subcores=NS
    )

    @functools.partial(
        pl.kernel,
        out_type=jax.ShapeDtypeStruct((NC * N_PAD, HALF), jnp.float32),
        mesh=mesh,
        scratch_types=[
            pltpu.VMEM((PCH, CK), jnp.int32),
            pltpu.VMEM((PCH, CK), jnp.int32),
            pltpu.VMEM((CK, HALF), jnp.float32),
            pltpu.VMEM((CK, HALF), jnp.float32),
            pltpu.VMEM((CK, HALF), jnp.float32),
            pltpu.VMEM((CK, HALF), jnp.float32),
            pltpu.VMEM_SHARED((N_PAD, HALF), jnp.float32),
            pltpu.SemaphoreType.DMA,
            pltpu.SemaphoreType.DMA,
            pltpu.SemaphoreType.DMA,
            pltpu.SemaphoreType.DMA,
            pltpu.SemaphoreType.DMA,
            pltpu.SemaphoreType.DMA,
            pltpu.SemaphoreType.DMA,
            pltpu.SemaphoreType.DMA,
        ],
    )
    def k(y_hbm, src_hbm, dst_hbm, out_hbm, src_v, dst_v,
          r0, r1, r2, r3, acc, g0, g1, g2, g3, s0, s1, s2, s3):
        c = lax.axis_index("c")
        s = lax.axis_index("s")
        w = c * NS + s
        rows = [r0, r1, r2, r3]
        gs = [g0, g1, g2, g3]
        ss = [s0, s1, s2, s3]

        def zero_row(i, _):
            for j in range(HALF // 16):
                r0[i, pl.ds(j * 16, 16)] = jnp.zeros((16,), jnp.float32)
            return 0
        lax.fori_loop(0, CK, zero_row, 0)
        base = s * ZR
        nfull = ZR // CK
        for k4 in range(nfull):
            pltpu.sync_copy(r0, acc.at[pl.ds(base + k4 * CK, CK)])
        rem = ZR - nfull * CK
        if rem:
            pltpu.sync_copy(r0.at[pl.ds(0, rem)],
                            acc.at[pl.ds(base + nfull * CK, rem)])
        plsc.subcore_barrier()

        def wait_g(rows_b, sem):
            pltpu.make_async_copy(y_hbm.at[src_v.at[0]], rows_b, sem).wait()

        def wait_s(rows_b, sem):
            pltpu.make_async_copy(rows_b, acc.at[dst_v.at[0]], sem).wait()

        for h in range(CHUNKS // PCH):
            pltpu.sync_copy(src_hbm.at[w, pl.ds(h * PCH, PCH)], src_v)
            pltpu.sync_copy(dst_hbm.at[s, pl.ds(h * PCH, PCH)], dst_v)
            pltpu.async_copy(y_hbm.at[src_v.at[0]], rows[0], gs[0])
            pltpu.async_copy(y_hbm.at[src_v.at[1]], rows[1], gs[1])

            def outer(i, _):
                for b in range(NBUF):
                    j = i * NBUF + b
                    wait_g(rows[b], gs[b])
                    pltpu.async_copy(rows[b], acc.at[dst_v.at[j]],
                                     ss[b], add=True)
                    b2 = (b + 2) % NBUF

                    @pl.when(j + 2 < PCH)
                    def _():
                        @pl.when(j >= 2)
                        def _():
                            wait_s(rows[b2], ss[b2])
                        pltpu.async_copy(y_hbm.at[src_v.at[j + 2]],
                                         rows[b2], gs[b2])
                return 0

            lax.fori_loop(0, PCH // NBUF, outer, 0)
            for b in range(NBUF):
                wait_s(rows[b], ss[b])
        plsc.subcore_barrier()

        pltpu.sync_copy(acc.at[pl.ds(base, ZR)],
                        out_hbm.at[pl.ds(c * N_PAD + base, ZR)])

    out = k(y_flat, src_w, dst_r)
    return out.reshape(NC, N_PAD, HALF)


def _k1_body(x_ref, w_ref, y_ref):
    r = jnp.dot(x_ref[...], w_ref[...], preferred_element_type=jnp.float32)
    y_ref[0] = r[:, :HALF]
    y_ref[1] = r[:, HALF:]


def _k1(x, W1a):
    return pl.pallas_call(
        _k1_body,
        grid=(NB,),
        in_specs=[
            pl.BlockSpec((BNODE, D_IN), lambda i: (i, 0)),
            pl.BlockSpec((D_IN, HID), lambda i: (0, 0)),
        ],
        out_specs=pl.BlockSpec((NC, BNODE, HALF), lambda i: (0, i, 0)),
        out_shape=jax.ShapeDtypeStruct((NC, N, HALF), jnp.float32),
    )(x, W1a)


def _k2_body(y_ref, a_ref, s1_ref, c1_ref, w1b_ref, b1b_ref, w2a_ref,
             x1_ref, y2_ref):
    h = jnp.concatenate([y_ref[0] + a_ref[0], y_ref[1] + a_ref[1]], axis=1)
    h = jnp.maximum(h * s1_ref[...] + c1_ref[...], 0.0)
    x1 = jnp.dot(h, w1b_ref[...], preferred_element_type=jnp.float32)
    x1 = jnp.maximum(x1 + b1b_ref[...], 0.0)
    x1_ref[...] = x1
    r2 = jnp.dot(x1, w2a_ref[...], preferred_element_type=jnp.float32)
    y2_ref[0] = r2[:, :HALF]
    y2_ref[1] = r2[:, HALF:]


def _k2(y, agg, s1, c1, W1b, b1b, W2a):
    return pl.pallas_call(
        _k2_body,
        grid=(NB,),
        in_specs=[
            pl.BlockSpec((NC, BNODE, HALF), lambda i: (0, i, 0)),
            pl.BlockSpec((NC, BNODE, HALF), lambda i: (0, i, 0)),
            pl.BlockSpec((1, HID), lambda i: (0, 0)),
            pl.BlockSpec((1, HID), lambda i: (0, 0)),
            pl.BlockSpec((HID, HID), lambda i: (0, 0)),
            pl.BlockSpec((1, HID), lambda i: (0, 0)),
            pl.BlockSpec((HID, HID), lambda i: (0, 0)),
        ],
        out_specs=[
            pl.BlockSpec((BNODE, HID), lambda i: (i, 0)),
            pl.BlockSpec((NC, BNODE, HALF), lambda i: (0, i, 0)),
        ],
        out_shape=[
            jax.ShapeDtypeStruct((N, HID), jnp.float32),
            jax.ShapeDtypeStruct((NC, N, HALF), jnp.float32),
        ],
    )(y, agg, s1, c1, W1b, b1b, W2a)


def _k3_body(y_ref, a_ref, s2_ref, c2_ref, w2b_ref, b2b_ref, x1_ref,
             wl_ref, bl_ref, batch_ref, out_ref):
    i = pl.program_id(0)
    h = jnp.concatenate([y_ref[0] + a_ref[0], y_ref[1] + a_ref[1]], axis=1)
    h = jnp.maximum(h * s2_ref[...] + c2_ref[...], 0.0)
    x2 = jnp.dot(h, w2b_ref[...], preferred_element_type=jnp.float32)
    x2 = jnp.maximum(x2 + b2b_ref[...], 0.0)
    p = (jnp.dot(x1_ref[...], wl_ref[:HID], preferred_element_type=jnp.float32)
         + jnp.dot(x2, wl_ref[HID:], preferred_element_type=jnp.float32))
    b = batch_ref[0, 0, :]
    onehot = (b[None, :] == lax.broadcasted_iota(jnp.int32, (G, BNODE), 0)
              ).astype(jnp.float32)
    contrib = jnp.dot(onehot, p, preferred_element_type=jnp.float32)

    @pl.when(i == 0)
    def _():
        out_ref[...] = jnp.broadcast_to(bl_ref[...], (G, 2))

    out_ref[...] += contrib


def _k3(y2, agg2, s2, c2, W2b, b2b, x1, Wl, bl, batch3):
    return pl.pallas_call(
        _k3_body,
        grid=(NB,),
        in_specs=[
            pl.BlockSpec((NC, BNODE, HALF), lambda i: (0, i, 0)),
            pl.BlockSpec((NC, BNODE, HALF), lambda i: (0, i, 0)),
            pl.BlockSpec((1, HID), lambda i: (0, 0)),
            pl.BlockSpec((1, HID), lambda i: (0, 0)),
            pl.BlockSpec((HID, HID), lambda i: (0, 0)),
            pl.BlockSpec((1, HID), lambda i: (0, 0)),
            pl.BlockSpec((BNODE, HID), lambda i: (i, 0)),
            pl.BlockSpec((2 * HID, 2), lambda i: (0, 0)),
            pl.BlockSpec((1, 2), lambda i: (0, 0)),
            pl.BlockSpec((1, 1, BNODE), lambda i: (i, 0, 0)),
        ],
        out_specs=pl.BlockSpec((G, 2), lambda i: (0, 0)),
        out_shape=jax.ShapeDtypeStruct((G, 2), jnp.float32),
    )(y2, agg2, s2, c2, W2b, b2b, x1, Wl, bl, batch3)


def kernel(x, edge_index, batch, W1a, b1a, g1, be1, W1b, b1b,
           W2a, b2a, g2, be2, W2b, b2b, Wl, bl):
    src = edge_index[0]
    dst = edge_index[1]

    pad = E_PAD - E
    src_p = jnp.concatenate([src, jnp.zeros((pad,), jnp.int32)])
    dst_p = jnp.concatenate([dst, jnp.full((pad,), N, jnp.int32)])
    src_r = src_p.reshape(NS, CHUNKS, CK)
    src_w = jnp.concatenate([src_r, src_r + N], axis=0).reshape(
        NC * NS, CHUNKS, CK)
    dst_r = dst_p.reshape(NS, CHUNKS, CK)
    batch3 = batch.reshape(NB, 1, BNODE)

    inv = 1.0 / jnp.sqrt(1.0 + BN_EPS)
    s1 = (g1 * inv).reshape(1, HID)
    c1 = (b1a * g1 * inv + be1).reshape(1, HID)
    s2 = (g2 * inv).reshape(1, HID)
    c2 = (b2a * g2 * inv + be2).reshape(1, HID)

    y = _k1(x, W1a)
    agg1 = _sc_segment_sum(y, src_w, dst_r)
    x1, y2 = _k2(y, agg1, s1, c1, W1b, b1b.reshape(1, HID), W2a)
    agg2 = _sc_segment_sum(y2, src_w, dst_r)
    out = _k3(y2, agg2, s2, c2, W2b, b2b.reshape(1, HID), x1,
              Wl, bl.reshape(1, 2), batch3)
    return out
